# baseline (speedup 1.0000x reference)
import functools

import jax
import jax.numpy as jnp
from jax import lax
from jax.experimental import pallas as pl
from jax.experimental.pallas import tpu as pltpu
from jax.experimental.pallas import tpu_sc as plsc

N_WIDTH = 32
N_NODES = 193
N_SAMPLES = 2048
N_ELEMENTS = 64
L = 16
NC, NS = 2, 16
NW = NC * NS
SPW = N_SAMPLES // NW
G = SPW // L

_C0 = (-0.5625, 0.5625, 0.0625, -0.0625)
_C1 = (1.6875, -0.5625, -1.6875, 0.5625)
_C2 = (-1.6875, -0.5625, 1.6875, 0.5625)
_C3 = (0.5625, 0.5625, -0.0625, -0.0625)


def _basis4(t):
    ps = []
    for a3, a2, a1, a0 in (_C0, _C1, _C2, _C3):
        ps.append(((a3 * t + a2) * t + a1) * t + a0)
    return ps


def _elem(xv):
    xs = xv * 192.0
    e = jnp.clip((xs / 3.0).astype(jnp.int32), 0, N_ELEMENTS - 1)
    b = e * 3
    t = (xs - (b.astype(jnp.float32) + 1.5)) / 1.5
    return b, t


def _make_kernel():
    mesh = plsc.VectorSubcoreMesh(core_axis_name="c", subcore_axis_name="s")

    @functools.partial(
        pl.kernel,
        mesh=mesh,
        compiler_params=pltpu.CompilerParams(needs_layout_passes=False),
        out_type=jax.ShapeDtypeStruct((N_SAMPLES,), jnp.float32),
        scratch_types=[
            pltpu.VMEM((SPW,), jnp.float32),
            pltpu.VMEM((N_WIDTH, N_NODES), jnp.float32),
            pltpu.VMEM((N_WIDTH, N_NODES), jnp.float32),
            pltpu.VMEM((SPW,), jnp.float32),
        ],
    )
    def kann(x_hbm, wi_hbm, wo_hbm, out_hbm, x_v, wi_v, wo_v, out_v):
        wid = lax.axis_index("s") * NC + lax.axis_index("c")
        base = wid * SPW
        pltpu.sync_copy(x_hbm.at[pl.ds(base, SPW)], x_v)
        pltpu.sync_copy(wi_hbm, wi_v)
        pltpu.sync_copy(wo_hbm, wo_v)

        for g in range(G):
            xv = x_v[pl.ds(g * L, L)]
            b1, t1c = _elem(xv)
            p1 = _basis4(t1c)

            def kbody(k, acc, b1=b1, p1=p1):
                kv = jnp.full((L,), 0, jnp.int32) + k
                t1k = p1[0] * plsc.load_gather(wi_v, [kv, b1])
                for j in range(1, 4):
                    t1k = t1k + p1[j] * plsc.load_gather(wi_v, [kv, b1 + j])
                b2, t2c = _elem(t1k)
                p2 = _basis4(t2c)
                r = p2[0] * plsc.load_gather(wo_v, [kv, b2])
                for j in range(1, 4):
                    r = r + p2[j] * plsc.load_gather(wo_v, [kv, b2 + j])
                return acc + r

            acc = lax.fori_loop(0, N_WIDTH, kbody, jnp.zeros((L,), jnp.float32))
            out_v[pl.ds(g * L, L)] = acc

        pltpu.sync_copy(out_v, out_hbm.at[pl.ds(base, SPW)])

    return kann


_kann = _make_kernel()


@jax.jit
def kernel(x, w_inner, w_outer):
    return _kann(x, w_inner, w_outer)


# speedup vs baseline: 1.0453x; 1.0453x over previous
import functools

import jax
import jax.numpy as jnp
from jax import lax
from jax.experimental import pallas as pl
from jax.experimental.pallas import tpu as pltpu
from jax.experimental.pallas import tpu_sc as plsc

N_WIDTH = 32
N_NODES = 193
N_SAMPLES = 2048
N_ELEMENTS = 64
L = 16
NC, NS = 2, 16
NW = NC * NS
SPW = N_SAMPLES // NW
G = SPW // L

_C0 = (-0.5625, 0.5625, 0.0625, -0.0625)
_C1 = (1.6875, -0.5625, -1.6875, 0.5625)
_C2 = (-1.6875, -0.5625, 1.6875, 0.5625)
_C3 = (0.5625, 0.5625, -0.0625, -0.0625)


def _basis4(t):
    ps = []
    for a3, a2, a1, a0 in (_C0, _C1, _C2, _C3):
        ps.append(((a3 * t + a2) * t + a1) * t + a0)
    return ps


def _elem(xv):
    xs = xv * 192.0
    e = jnp.clip((xs * (1.0 / 3.0)).astype(jnp.int32), 0, N_ELEMENTS - 1)
    b = e * 3
    t = (xs - (b.astype(jnp.float32) + 1.5)) * (1.0 / 1.5)
    return b, t


def _make_kernel():
    mesh = plsc.VectorSubcoreMesh(core_axis_name="c", subcore_axis_name="s")

    @functools.partial(
        pl.kernel,
        mesh=mesh,
        compiler_params=pltpu.CompilerParams(needs_layout_passes=False),
        out_type=jax.ShapeDtypeStruct((N_SAMPLES,), jnp.float32),
        scratch_types=[
            pltpu.VMEM((SPW,), jnp.float32),
            pltpu.VMEM((N_WIDTH * N_NODES,), jnp.float32),
            pltpu.VMEM((N_WIDTH * N_NODES,), jnp.float32),
            pltpu.VMEM((SPW,), jnp.float32),
        ],
    )
    def kann(x_hbm, wi_hbm, wo_hbm, out_hbm, x_v, wi_v, wo_v, out_v):
        wid = lax.axis_index("s") * NC + lax.axis_index("c")
        base = wid * SPW
        pltpu.sync_copy(x_hbm.at[pl.ds(base, SPW)], x_v)
        pltpu.sync_copy(wi_hbm, wi_v)
        pltpu.sync_copy(wo_hbm, wo_v)

        for g in range(G):
            xv = x_v[pl.ds(g * L, L)]
            b1, t1c = _elem(xv)
            p1 = _basis4(t1c)

            @plsc.parallel_loop(0, N_WIDTH, unroll=2, carry=jnp.zeros((L,), jnp.float32))
            def acc(k, acc, b1=b1, p1=p1):
                kb = k * N_NODES
                idx1 = b1 + kb
                t1k = p1[0] * plsc.load_gather(wi_v, [idx1])
                for j in range(1, 4):
                    t1k = t1k + p1[j] * plsc.load_gather(wi_v, [idx1 + j])
                b2, t2c = _elem(t1k)
                p2 = _basis4(t2c)
                idx2 = b2 + kb
                r = p2[0] * plsc.load_gather(wo_v, [idx2])
                for j in range(1, 4):
                    r = r + p2[j] * plsc.load_gather(wo_v, [idx2 + j])
                return acc + r

            out_v[pl.ds(g * L, L)] = acc

        pltpu.sync_copy(out_v, out_hbm.at[pl.ds(base, SPW)])

    return kann


_kann = _make_kernel()


@jax.jit
def kernel(x, w_inner, w_outer):
    return _kann(x, w_inner.reshape(-1), w_outer.reshape(-1))


# speedup vs baseline: 1.0795x; 1.0326x over previous
import functools

import jax
import jax.numpy as jnp
from jax import lax
from jax.experimental import pallas as pl
from jax.experimental.pallas import tpu as pltpu
from jax.experimental.pallas import tpu_sc as plsc

N_WIDTH = 32
N_NODES = 193
N_SAMPLES = 2048
N_ELEMENTS = 64
L = 16
NC, NS = 2, 16
NW = NC * NS
SPW = N_SAMPLES // NW
G = SPW // L

_C0 = (-0.5625, 0.5625, 0.0625, -0.0625)
_C1 = (1.6875, -0.5625, -1.6875, 0.5625)
_C2 = (-1.6875, -0.5625, 1.6875, 0.5625)
_C3 = (0.5625, 0.5625, -0.0625, -0.0625)


def _basis4(t):
    ps = []
    for a3, a2, a1, a0 in (_C0, _C1, _C2, _C3):
        ps.append(((a3 * t + a2) * t + a1) * t + a0)
    return ps


def _elem(xv):
    xs = xv * 192.0
    e = jnp.clip((xs * (1.0 / 3.0)).astype(jnp.int32), 0, N_ELEMENTS - 1)
    b = e * 3
    t = (xs - (b.astype(jnp.float32) + 1.5)) * (1.0 / 1.5)
    return b, t


def _make_kernel():
    mesh = plsc.VectorSubcoreMesh(core_axis_name="c", subcore_axis_name="s")

    @functools.partial(
        pl.kernel,
        mesh=mesh,
        compiler_params=pltpu.CompilerParams(needs_layout_passes=False),
        out_type=jax.ShapeDtypeStruct((N_SAMPLES,), jnp.float32),
        scratch_types=[
            pltpu.VMEM((SPW,), jnp.float32),
            pltpu.VMEM((N_WIDTH * N_NODES,), jnp.float32),
            pltpu.VMEM((N_WIDTH * N_NODES,), jnp.float32),
            pltpu.VMEM((SPW,), jnp.float32),
        ],
    )
    def kann(x_hbm, wi_hbm, wo_hbm, out_hbm, x_v, wi_v, wo_v, out_v):
        wid = lax.axis_index("s") * NC + lax.axis_index("c")
        base = wid * SPW
        pltpu.sync_copy(x_hbm.at[pl.ds(base, SPW)], x_v)
        pltpu.sync_copy(wi_hbm, wi_v)
        pltpu.sync_copy(wo_hbm, wo_v)

        b1s, p1s = [], []
        for g in range(G):
            xv = x_v[pl.ds(g * L, L)]
            b1, t1c = _elem(xv)
            b1s.append(b1)
            p1s.append(_basis4(t1c))

        def kbody(k, accs):
            kb = k * N_NODES
            out = []
            for g in range(G):
                b1, p1 = b1s[g], p1s[g]
                idx1 = b1 + kb
                t1k = p1[0] * plsc.load_gather(wi_v, [idx1])
                for j in range(1, 4):
                    t1k = t1k + p1[j] * plsc.load_gather(wi_v, [idx1 + j])
                b2, t2c = _elem(t1k)
                p2 = _basis4(t2c)
                idx2 = b2 + kb
                r = p2[0] * plsc.load_gather(wo_v, [idx2])
                for j in range(1, 4):
                    r = r + p2[j] * plsc.load_gather(wo_v, [idx2 + j])
                out.append(accs[g] + r)
            return tuple(out)

        accs = lax.fori_loop(
            0, N_WIDTH, kbody, tuple(jnp.zeros((L,), jnp.float32) for _ in range(G))
        )
        for g in range(G):
            out_v[pl.ds(g * L, L)] = accs[g]

        pltpu.sync_copy(out_v, out_hbm.at[pl.ds(base, SPW)])

    return kann


_kann = _make_kernel()


@jax.jit
def kernel(x, w_inner, w_outer):
    return _kann(x, w_inner.reshape(-1), w_outer.reshape(-1))


# speedup vs baseline: 1.0977x; 1.0169x over previous
import functools

import jax
import jax.numpy as jnp
from jax import lax
from jax.experimental import pallas as pl
from jax.experimental.pallas import tpu as pltpu
from jax.experimental.pallas import tpu_sc as plsc

N_WIDTH = 32
N_NODES = 193
N_SAMPLES = 2048
N_ELEMENTS = 64
L = 16
NC, NS = 2, 16
NW = NC * NS
SPW = N_SAMPLES // NW
G = SPW // L

_C0 = (-0.5625, 0.5625, 0.0625, -0.0625)
_C1 = (1.6875, -0.5625, -1.6875, 0.5625)
_C2 = (-1.6875, -0.5625, 1.6875, 0.5625)
_C3 = (0.5625, 0.5625, -0.0625, -0.0625)


def _basis4(t):
    ps = []
    for a3, a2, a1, a0 in (_C0, _C1, _C2, _C3):
        ps.append(((a3 * t + a2) * t + a1) * t + a0)
    return ps


def _elem(xv):
    xs = xv * 192.0
    e = jnp.clip((xs * (1.0 / 3.0)).astype(jnp.int32), 0, N_ELEMENTS - 1)
    b = e * 3
    t = (xs - (b.astype(jnp.float32) + 1.5)) * (1.0 / 1.5)
    return b, t


def _make_kernel():
    mesh = plsc.VectorSubcoreMesh(core_axis_name="c", subcore_axis_name="s")

    @functools.partial(
        pl.kernel,
        mesh=mesh,
        compiler_params=pltpu.CompilerParams(needs_layout_passes=False),
        out_type=jax.ShapeDtypeStruct((N_SAMPLES,), jnp.float32),
        scratch_types=[
            pltpu.VMEM((SPW,), jnp.float32),
            pltpu.VMEM((N_WIDTH * N_NODES,), jnp.float32),
            pltpu.VMEM((N_WIDTH * N_NODES,), jnp.float32),
            pltpu.VMEM((SPW,), jnp.float32),
        ],
    )
    def kann(x_hbm, wi_hbm, wo_hbm, out_hbm, x_v, wi_v, wo_v, out_v):
        wid = lax.axis_index("s") * NC + lax.axis_index("c")
        base = wid * SPW
        pltpu.sync_copy(x_hbm.at[pl.ds(base, SPW)], x_v)
        pltpu.sync_copy(wi_hbm, wi_v)
        pltpu.sync_copy(wo_hbm, wo_v)

        b1s, p1s = [], []
        for g in range(G):
            xv = x_v[pl.ds(g * L, L)]
            b1, t1c = _elem(xv)
            b1s.append(b1)
            p1s.append(_basis4(t1c))

        def kbody(k, accs):
            kb = k * N_NODES
            out = []
            for g in range(G):
                b1, p1 = b1s[g], p1s[g]
                idx1 = b1 + kb
                t1k = p1[0] * plsc.load_gather(wi_v, [idx1])
                for j in range(1, 4):
                    t1k = t1k + p1[j] * plsc.load_gather(wi_v, [idx1 + j])
                b2, t2c = _elem(t1k)
                idx2 = b2 + kb
                g0 = plsc.load_gather(wo_v, [idx2])
                g1 = plsc.load_gather(wo_v, [idx2 + 1])
                g2 = plsc.load_gather(wo_v, [idx2 + 2])
                g3 = plsc.load_gather(wo_v, [idx2 + 3])
                tt = t2c * t2c
                u2 = 0.5625 * tt - 0.0625
                v2 = 0.5625 - 0.5625 * tt
                tu2 = t2c * u2
                w3 = 3.0 * (t2c * v2)
                r = u2 * (g0 + g3) + tu2 * (g3 - g0)
                r = r + v2 * (g1 + g2) + w3 * (g2 - g1)
                out.append(accs[g] + r)
            return tuple(out)

        accs = lax.fori_loop(
            0, N_WIDTH, kbody, tuple(jnp.zeros((L,), jnp.float32) for _ in range(G))
        )
        for g in range(G):
            out_v[pl.ds(g * L, L)] = accs[g]

        pltpu.sync_copy(out_v, out_hbm.at[pl.ds(base, SPW)])

    return kann


_kann = _make_kernel()


@jax.jit
def kernel(x, w_inner, w_outer):
    return _kann(x, w_inner.reshape(-1), w_outer.reshape(-1))


# speedup vs baseline: 1.1434x; 1.0416x over previous
import functools

import jax
import jax.numpy as jnp
from jax import lax
from jax.experimental import pallas as pl
from jax.experimental.pallas import tpu as pltpu
from jax.experimental.pallas import tpu_sc as plsc

N_WIDTH = 32
N_NODES = 193
N_SAMPLES = 2048
N_ELEMENTS = 64
L = 16
NC, NS = 2, 16
NW = NC * NS
SPW = N_SAMPLES // NW
G = SPW // L

_C0 = (-0.5625, 0.5625, 0.0625, -0.0625)
_C1 = (1.6875, -0.5625, -1.6875, 0.5625)
_C2 = (-1.6875, -0.5625, 1.6875, 0.5625)
_C3 = (0.5625, 0.5625, -0.0625, -0.0625)


def _basis4(t):
    ps = []
    for a3, a2, a1, a0 in (_C0, _C1, _C2, _C3):
        ps.append(((a3 * t + a2) * t + a1) * t + a0)
    return ps


def _elem(xv):
    xs = xv * 192.0
    e = jnp.clip((xs * (1.0 / 3.0)).astype(jnp.int32), 0, N_ELEMENTS - 1)
    b = e * 3
    t = (xs - (b.astype(jnp.float32) + 1.5)) * (1.0 / 1.5)
    return b, t


def _make_kernel():
    mesh = plsc.VectorSubcoreMesh(core_axis_name="c", subcore_axis_name="s")

    @functools.partial(
        pl.kernel,
        mesh=mesh,
        compiler_params=pltpu.CompilerParams(needs_layout_passes=False),
        out_type=jax.ShapeDtypeStruct((N_SAMPLES,), jnp.float32),
        scratch_types=[
            pltpu.VMEM((SPW,), jnp.float32),
            pltpu.VMEM((N_WIDTH * N_NODES,), jnp.float32),
            pltpu.VMEM((N_WIDTH * N_NODES,), jnp.float32),
            pltpu.VMEM((SPW,), jnp.float32),
            pltpu.SemaphoreType.DMA,
            pltpu.SemaphoreType.DMA,
            pltpu.SemaphoreType.DMA,
        ],
    )
    def kann(x_hbm, wi_hbm, wo_hbm, out_hbm, x_v, wi_v, wo_v, out_v,
             sem_x, sem_wi, sem_wo):
        wid = lax.axis_index("s") * NC + lax.axis_index("c")
        base = wid * SPW
        cp_x = pltpu.async_copy(x_hbm.at[pl.ds(base, SPW)], x_v, sem_x)
        cp_wi = pltpu.async_copy(wi_hbm, wi_v, sem_wi)
        cp_wo = pltpu.async_copy(wo_hbm, wo_v, sem_wo)
        cp_x.wait()

        b1s, p1s = [], []
        for g in range(G):
            xv = x_v[pl.ds(g * L, L)]
            b1, t1c = _elem(xv)
            b1s.append(b1)
            p1s.append(_basis4(t1c))
        cp_wi.wait()
        cp_wo.wait()

        def kbody(k, accs):
            kb = k * N_NODES
            out = []
            for g in range(G):
                b1, p1 = b1s[g], p1s[g]
                idx1 = b1 + kb
                t1k = p1[0] * plsc.load_gather(wi_v, [idx1])
                for j in range(1, 4):
                    t1k = t1k + p1[j] * plsc.load_gather(wi_v, [idx1 + j])
                b2, t2c = _elem(t1k)
                idx2 = b2 + kb
                g0 = plsc.load_gather(wo_v, [idx2])
                g1 = plsc.load_gather(wo_v, [idx2 + 1])
                g2 = plsc.load_gather(wo_v, [idx2 + 2])
                g3 = plsc.load_gather(wo_v, [idx2 + 3])
                tt = t2c * t2c
                u2 = 0.5625 * tt - 0.0625
                v2 = 0.5625 - 0.5625 * tt
                tu2 = t2c * u2
                w3 = 3.0 * (t2c * v2)
                r = u2 * (g0 + g3) + tu2 * (g3 - g0)
                r = r + v2 * (g1 + g2) + w3 * (g2 - g1)
                out.append(accs[g] + r)
            return tuple(out)

        accs = lax.fori_loop(
            0, N_WIDTH, kbody, tuple(jnp.zeros((L,), jnp.float32) for _ in range(G))
        )
        for g in range(G):
            out_v[pl.ds(g * L, L)] = accs[g]

        pltpu.sync_copy(out_v, out_hbm.at[pl.ds(base, SPW)])

    return kann


_kann = _make_kernel()


@jax.jit
def kernel(x, w_inner, w_outer):
    return _kann(x, w_inner.reshape(-1), w_outer.reshape(-1))
